# baseline (speedup 1.0000x reference)
import functools

import jax
import jax.numpy as jnp
from jax import lax
from jax.experimental import pallas as pl
from jax.experimental.pallas import tpu as pltpu
from jax.experimental.pallas import tpu_sc as plsc

D_ROW = 4096
D_COL = 64
D_ALL = D_ROW * D_COL
BATCH = 128
K = D_ROW * D_COL

NUM_CORES = 2
NUM_SUBCORES = 16
KT = K // NUM_SUBCORES
CHUNK = 128
FIRE = 8
NCHUNK_T = KT // CHUNK
STEPS = BATCH // NUM_CORES


def _mapper_kernel(x_hbm, filt_hbm, ridx_hbm, fidx_hbm, out_hbm,
                   ridx_v, fidx_v, gft_v, gbuf, xrow_sh, gsem):
    c = lax.axis_index("c")
    s = lax.axis_index("s")
    rowbase = s * NCHUNK_T
    kbase = s * KT

    pltpu.sync_copy(ridx_hbm.at[pl.ds(rowbase, NCHUNK_T)], ridx_v)
    pltpu.sync_copy(fidx_hbm.at[pl.ds(rowbase, NCHUNK_T)], fidx_v)

    def fgather(jj, carry):
        cps = []
        for u in range(FIRE):
            j = jj * FIRE + u
            off = pl.multiple_of(j * CHUNK, CHUNK)
            cps.append(pltpu.async_copy(
                filt_hbm.at[fidx_v.at[j]], gft_v.at[pl.ds(off, CHUNK)], gsem))
        for cp in cps:
            cp.wait()
        return carry
    lax.fori_loop(0, NCHUNK_T // FIRE, fgather, 0)

    def body(t, carry):
        b = t * NUM_CORES + c

        @pl.when(s == 0)
        def _stage():
            pltpu.sync_copy(x_hbm.at[pl.ds(b * D_ALL, D_ALL)], xrow_sh)
        plsc.subcore_barrier()

        def ggather(jj, inner):
            cps = []
            for u in range(FIRE):
                j = jj * FIRE + u
                off = pl.multiple_of(j * CHUNK, CHUNK)
                cps.append(pltpu.async_copy(
                    xrow_sh.at[ridx_v.at[j]], gbuf.at[pl.ds(off, CHUNK)], gsem))
            for cp in cps:
                cp.wait()
            return inner
        lax.fori_loop(0, NCHUNK_T // FIRE, ggather, 0)

        def mul(i, inner):
            off = pl.multiple_of(i * 16, 16)
            sl = pl.ds(off, 16)
            gbuf[sl] = gbuf[sl] * gft_v[sl]
            return inner
        lax.fori_loop(0, KT // 16, mul, 0)

        pltpu.sync_copy(gbuf, out_hbm.at[pl.ds(b * K + kbase, KT)])
        plsc.subcore_barrier()
        return carry
    lax.fori_loop(0, STEPS, body, 0)


@jax.jit
def _mapper(x_flat, filt_flat, ridx2, fidx2):
    mesh = plsc.VectorSubcoreMesh(
        core_axis_name="c", subcore_axis_name="s",
        num_cores=NUM_CORES, num_subcores=NUM_SUBCORES)
    f = functools.partial(
        pl.kernel,
        out_type=jax.ShapeDtypeStruct((BATCH * K,), jnp.float32),
        mesh=mesh,
        scratch_types=[
            pltpu.VMEM((NCHUNK_T, CHUNK), jnp.int32),
            pltpu.VMEM((NCHUNK_T, CHUNK), jnp.int32),
            pltpu.VMEM((KT,), jnp.float32),
            pltpu.VMEM((KT,), jnp.float32),
            pltpu.VMEM_SHARED((D_ALL,), jnp.float32),
            pltpu.SemaphoreType.DMA,
        ],
    )(_mapper_kernel)
    return f(x_flat, filt_flat, ridx2, fidx2)


def kernel(x, filters, ind):
    x = x.reshape(-1, D_ALL)
    jnd = ind.T.astype(jnp.int32)
    ridx = jnd.reshape(-1)
    fidx = (jnd + jnp.arange(D_COL, dtype=jnp.int32)[None, :] * D_ALL).reshape(-1)
    ridx2 = ridx.reshape(K // CHUNK, CHUNK)
    fidx2 = fidx.reshape(K // CHUNK, CHUNK)
    out = _mapper(x.reshape(-1), filters.reshape(-1), ridx2, fidx2)
    return out.reshape(BATCH, D_ROW, D_COL)


# speedup vs baseline: 1.5051x; 1.5051x over previous
import functools

import jax
import jax.numpy as jnp
from jax import lax
from jax.experimental import pallas as pl
from jax.experimental.pallas import tpu as pltpu
from jax.experimental.pallas import tpu_sc as plsc

D_ROW = 4096
D_COL = 64
D_ALL = D_ROW * D_COL
BATCH = 128
K = D_ROW * D_COL

NUM_CORES = 2
NUM_SUBCORES = 16
KT = K // NUM_SUBCORES
CHUNK = 2048
NSTREAM = KT // CHUNK
MUL_UNROLL = 8
STEPS = BATCH // NUM_CORES
ROW_BYTES = D_ALL * 4


def _mapper_kernel(x_hbm, filt_hbm, ridx_hbm, fidx_hbm, out_hbm,
                   ridx_v, fidx_v, gft_v, gbuf, sh0, sh1, gsem, rsem):
    c = lax.axis_index("c")
    s = lax.axis_index("s")
    kbase = s * KT

    pltpu.sync_copy(ridx_hbm.at[pl.ds(kbase, KT)], ridx_v)
    pltpu.sync_copy(fidx_hbm.at[pl.ds(kbase, KT)], fidx_v)

    fcps = []
    for j in range(NSTREAM):
        sl = pl.ds(j * CHUNK, CHUNK)
        fcps.append(pltpu.async_copy(filt_hbm.at[fidx_v.at[sl]],
                                     gft_v.at[sl], gsem))
    for cp in fcps:
        cp.wait()

    @pl.when(s == 0)
    def _prime():
        pltpu.async_copy(x_hbm.at[pl.ds(c * D_ALL, D_ALL)], sh0, rsem)

    def step(t, sh, sh_next):
        b = t * NUM_CORES + c

        @pl.when(s == 0)
        def _wait_row():
            pltpu.make_async_copy(x_hbm.at[pl.ds(b * D_ALL, D_ALL)],
                                  sh, rsem).wait()
        plsc.subcore_barrier()

        @pl.when((s == 0) & (t + 1 < STEPS))
        def _stage_next():
            bn = (t + 1) * NUM_CORES + c
            pltpu.async_copy(x_hbm.at[pl.ds(bn * D_ALL, D_ALL)],
                             sh_next, rsem)

        gcps = []
        for j in range(NSTREAM):
            sl = pl.ds(j * CHUNK, CHUNK)
            gcps.append(pltpu.async_copy(sh.at[ridx_v.at[sl]],
                                         gbuf.at[sl], gsem))
        for cp in gcps:
            cp.wait()

        def mul(i, inner):
            base = pl.multiple_of(i * (16 * MUL_UNROLL), 16 * MUL_UNROLL)
            for u in range(MUL_UNROLL):
                sl = pl.ds(base + u * 16, 16)
                gbuf[sl] = gbuf[sl] * gft_v[sl]
            return inner
        lax.fori_loop(0, KT // (16 * MUL_UNROLL), mul, 0)

        pltpu.sync_copy(gbuf, out_hbm.at[pl.ds(b * K + kbase, KT)])

    def body(t2, carry):
        step(t2 * 2, sh0, sh1)
        step(t2 * 2 + 1, sh1, sh0)
        return carry
    lax.fori_loop(0, STEPS // 2, body, 0)


@jax.jit
def _mapper(x_flat, filt_flat, ridx, fidx):
    mesh = plsc.VectorSubcoreMesh(
        core_axis_name="c", subcore_axis_name="s",
        num_cores=NUM_CORES, num_subcores=NUM_SUBCORES)
    f = functools.partial(
        pl.kernel,
        out_type=jax.ShapeDtypeStruct((BATCH * K,), jnp.float32),
        mesh=mesh,
        scratch_types=[
            pltpu.VMEM((KT,), jnp.int32),
            pltpu.VMEM((KT,), jnp.int32),
            pltpu.VMEM((KT,), jnp.float32),
            pltpu.VMEM((KT,), jnp.float32),
            pltpu.VMEM_SHARED((D_ALL,), jnp.float32),
            pltpu.VMEM_SHARED((D_ALL,), jnp.float32),
            pltpu.SemaphoreType.DMA,
            pltpu.SemaphoreType.DMA,
        ],
    )(_mapper_kernel)
    return f(x_flat, filt_flat, ridx, fidx)


def kernel(x, filters, ind):
    x = x.reshape(-1, D_ALL)
    jnd = ind.T.astype(jnp.int32)
    ridx = jnd.reshape(-1)
    fidx = (jnd + jnp.arange(D_COL, dtype=jnp.int32)[None, :] * D_ALL).reshape(-1)
    out = _mapper(x.reshape(-1), filters.reshape(-1), ridx, fidx)
    return out.reshape(BATCH, D_ROW, D_COL)


# speedup vs baseline: 2.0326x; 1.3505x over previous
import functools

import jax
import jax.numpy as jnp
from jax import lax
from jax.experimental import pallas as pl
from jax.experimental.pallas import tpu as pltpu
from jax.experimental.pallas import tpu_sc as plsc

D_ROW = 4096
D_COL = 64
D_ALL = D_ROW * D_COL
BATCH = 128
K = D_ROW * D_COL

NUM_CORES = 2
NUM_SUBCORES = 16
NW = NUM_CORES * NUM_SUBCORES
KW = K // NW
CH = 128
NCH = KW // CH
FCH = 2048
CB = 2048


def _t1_body(x_ref, o_ref):
    o_ref[...] = x_ref[...].T


def _t2_body(y_ref, gft_ref, o_ref):
    o_ref[...] = y_ref[...].T * gft_ref[...][None, :]


def _gather_kernel(xT, filt, ridx, fidx, y_out, gft_out,
                   idx_v, gft_v, rows0, rows1, gsem, wsem):
    c = lax.axis_index("c")
    s = lax.axis_index("s")
    w = s * NUM_CORES + c
    kw = w * KW

    pltpu.sync_copy(fidx.at[pl.ds(kw, KW)], idx_v)
    fcps = []
    for q in range(KW // FCH):
        sl = pl.ds(q * FCH, FCH)
        fcps.append(pltpu.async_copy(filt.at[idx_v.at[sl]], gft_v.at[sl], gsem))
    for cp in fcps:
        cp.wait()
    pltpu.sync_copy(gft_v, gft_out.at[pl.ds(kw, KW)])

    pltpu.sync_copy(ridx.at[pl.ds(kw, KW)], idx_v)
    bufs = (rows0, rows1)

    def idx_sl(n):
        return idx_v.at[pl.ds(n * CH, CH)]

    def y_sl(n):
        return y_out.at[pl.ds(kw + n * CH, CH)]

    pltpu.async_copy(xT.at[idx_sl(0)], rows0, gsem)

    def step(n, buf, other):
        pltpu.make_async_copy(xT.at[idx_sl(n)], buf, gsem).wait()

        @pl.when(n + 1 < NCH)
        def _next():
            @pl.when(n >= 1)
            def _drain_prev_write():
                pltpu.make_async_copy(other, y_sl(n - 1), wsem).wait()
            pltpu.async_copy(xT.at[idx_sl(n + 1)], other, gsem)

        pltpu.async_copy(buf, y_sl(n), wsem)

    def body(m, carry):
        n = m * 2
        step(n, rows0, rows1)
        step(n + 1, rows1, rows0)
        return carry
    lax.fori_loop(0, NCH // 2, body, 0)

    pltpu.make_async_copy(rows0, y_sl(NCH - 2), wsem).wait()
    pltpu.make_async_copy(rows1, y_sl(NCH - 1), wsem).wait()


@jax.jit
def _mapper(x, filt_flat, ridx, fidx):
    xT = pl.pallas_call(
        _t1_body,
        grid=(D_ALL // CB,),
        in_specs=[pl.BlockSpec((BATCH, CB), lambda i: (0, i))],
        out_specs=pl.BlockSpec((CB, BATCH), lambda i: (i, 0)),
        out_shape=jax.ShapeDtypeStruct((D_ALL, BATCH), jnp.float32),
    )(x)

    mesh = plsc.VectorSubcoreMesh(
        core_axis_name="c", subcore_axis_name="s",
        num_cores=NUM_CORES, num_subcores=NUM_SUBCORES)
    y, gft = functools.partial(
        pl.kernel,
        out_type=(jax.ShapeDtypeStruct((K, BATCH), jnp.float32),
                  jax.ShapeDtypeStruct((K,), jnp.float32)),
        mesh=mesh,
        scratch_types=[
            pltpu.VMEM((KW,), jnp.int32),
            pltpu.VMEM((KW,), jnp.float32),
            pltpu.VMEM((CH, BATCH), jnp.float32),
            pltpu.VMEM((CH, BATCH), jnp.float32),
            pltpu.SemaphoreType.DMA,
            pltpu.SemaphoreType.DMA,
        ],
    )(_gather_kernel)(xT, filt_flat, ridx, fidx)

    out = pl.pallas_call(
        _t2_body,
        grid=(K // CB,),
        in_specs=[pl.BlockSpec((CB, BATCH), lambda i: (i, 0)),
                  pl.BlockSpec((CB,), lambda i: (i,))],
        out_specs=pl.BlockSpec((BATCH, CB), lambda i: (0, i)),
        out_shape=jax.ShapeDtypeStruct((BATCH, K), jnp.float32),
    )(y, gft)
    return out


def kernel(x, filters, ind):
    x = x.reshape(-1, D_ALL)
    jnd = ind.T.astype(jnp.int32)
    ridx = jnd.reshape(-1)
    fidx = (jnd + jnp.arange(D_COL, dtype=jnp.int32)[None, :] * D_ALL).reshape(-1)
    out = _mapper(x, filters.reshape(-1), ridx, fidx)
    return out.reshape(BATCH, D_ROW, D_COL)


# speedup vs baseline: 2.0389x; 1.0031x over previous
import functools

import jax
import jax.numpy as jnp
from jax import lax
from jax.experimental import pallas as pl
from jax.experimental.pallas import tpu as pltpu
from jax.experimental.pallas import tpu_sc as plsc

D_ROW = 4096
D_COL = 64
D_ALL = D_ROW * D_COL
BATCH = 128
K = D_ROW * D_COL

NUM_CORES = 2
NUM_SUBCORES = 16
NW = NUM_CORES * NUM_SUBCORES
KW = K // NW
CH = 128
NCH = KW // CH
FCH = 2048
CB = 2048


def _t1_body(x_ref, o_ref):
    o_ref[...] = x_ref[...].T


def _t2_body(y_ref, gft_ref, o_ref):
    o_ref[...] = y_ref[...].T * gft_ref[...][None, :]


def _gather_kernel(xT, filt, ridx, fidx, y_out, gft_out,
                   idx_v, gft_v, rows0, rows1, gsem, wsem):
    c = lax.axis_index("c")
    s = lax.axis_index("s")
    w = s * NUM_CORES + c
    kw = w * KW

    pltpu.sync_copy(fidx.at[pl.ds(kw, KW)], idx_v)
    fcps = []
    for q in range(KW // FCH):
        sl = pl.ds(q * FCH, FCH)
        fcps.append(pltpu.async_copy(filt.at[idx_v.at[sl]], gft_v.at[sl], gsem))
    for cp in fcps:
        cp.wait()
    pltpu.sync_copy(gft_v, gft_out.at[pl.ds(kw, KW)])

    pltpu.sync_copy(ridx.at[pl.ds(kw, KW)], idx_v)
    bufs = (rows0, rows1)

    def idx_sl(n):
        return idx_v.at[pl.ds(n * CH, CH)]

    def y_sl(n):
        return y_out.at[pl.ds(kw + n * CH, CH)]

    pltpu.async_copy(xT.at[idx_sl(0)], rows0, gsem)

    def step(n, buf, other):
        pltpu.make_async_copy(xT.at[idx_sl(n)], buf, gsem).wait()

        @pl.when(n + 1 < NCH)
        def _next():
            @pl.when(n >= 1)
            def _drain_prev_write():
                pltpu.make_async_copy(other, y_sl(n - 1), wsem).wait()
            pltpu.async_copy(xT.at[idx_sl(n + 1)], other, gsem)

        pltpu.async_copy(buf, y_sl(n), wsem)

    def body(m, carry):
        n = m * 2
        step(n, rows0, rows1)
        step(n + 1, rows1, rows0)
        return carry
    lax.fori_loop(0, NCH // 2, body, 0)

    pltpu.make_async_copy(rows0, y_sl(NCH - 2), wsem).wait()
    pltpu.make_async_copy(rows1, y_sl(NCH - 1), wsem).wait()


@jax.jit
def _mapper(x, filt_flat, ridx, fidx):
    xT = pl.pallas_call(
        _t1_body,
        grid=(D_ALL // CB,),
        in_specs=[pl.BlockSpec((BATCH, CB), lambda i: (0, i))],
        out_specs=pl.BlockSpec((CB, BATCH), lambda i: (i, 0)),
        out_shape=jax.ShapeDtypeStruct((D_ALL, BATCH), jnp.float32),
    )(x)

    mesh = plsc.VectorSubcoreMesh(
        core_axis_name="c", subcore_axis_name="s",
        num_cores=NUM_CORES, num_subcores=NUM_SUBCORES)
    y, gft = functools.partial(
        pl.kernel,
        out_type=(jax.ShapeDtypeStruct((K, BATCH), jnp.float32),
                  jax.ShapeDtypeStruct((K,), jnp.float32)),
        mesh=mesh,
        compiler_params=pltpu.CompilerParams(use_tc_tiling_on_sc=True),
        scratch_types=[
            pltpu.VMEM((KW,), jnp.int32),
            pltpu.VMEM((KW,), jnp.float32),
            pltpu.VMEM((CH, BATCH), jnp.float32),
            pltpu.VMEM((CH, BATCH), jnp.float32),
            pltpu.SemaphoreType.DMA,
            pltpu.SemaphoreType.DMA,
        ],
    )(_gather_kernel)(xT, filt_flat, ridx, fidx)

    out = pl.pallas_call(
        _t2_body,
        grid=(K // CB,),
        in_specs=[pl.BlockSpec((CB, BATCH), lambda i: (i, 0)),
                  pl.BlockSpec((CB,), lambda i: (i,))],
        out_specs=pl.BlockSpec((BATCH, CB), lambda i: (0, i)),
        out_shape=jax.ShapeDtypeStruct((BATCH, K), jnp.float32),
    )(y, gft)
    return out


def kernel(x, filters, ind):
    x = x.reshape(-1, D_ALL)
    jnd = ind.T.astype(jnp.int32)
    ridx = jnd.reshape(-1)
    fidx = (jnd + jnp.arange(D_COL, dtype=jnp.int32)[None, :] * D_ALL).reshape(-1)
    out = _mapper(x, filters.reshape(-1), ridx, fidx)
    return out.reshape(BATCH, D_ROW, D_COL)


# speedup vs baseline: 2.6781x; 1.3135x over previous
import functools

import jax
import jax.numpy as jnp
from jax import lax
from jax.experimental import pallas as pl
from jax.experimental.pallas import tpu as pltpu
from jax.experimental.pallas import tpu_sc as plsc

D_ROW = 4096
D_COL = 64
D_ALL = D_ROW * D_COL
BATCH = 128
K = D_ROW * D_COL
NQ = 4
KQ = K // NQ

NUM_CORES = 2
NUM_SUBCORES = 16
NW = NUM_CORES * NUM_SUBCORES
KW = K // NW
KWQ = KQ // NW
CH = 128
NCHQ = KWQ // CH
FCH = 2048
CB = 2048


def _t1_body(x_ref, o_ref):
    o_ref[...] = x_ref[...].T


def _t2_body0(y_ref, gft_ref, o_ref):
    o_ref[...] = y_ref[...].T * gft_ref[...][None, :]


def _t2_body1(y_ref, gft_ref, acc_ref, o_ref):
    del acc_ref
    o_ref[...] = y_ref[...].T * gft_ref[...][None, :]


def _gft_kernel(filt, fidx, gft_out, idx_v, gft_v, gsem):
    c = lax.axis_index("c")
    s = lax.axis_index("s")
    kw = (s * NUM_CORES + c) * KW
    pltpu.sync_copy(fidx.at[pl.ds(kw, KW)], idx_v)
    fcps = []
    for q in range(KW // FCH):
        sl = pl.ds(q * FCH, FCH)
        fcps.append(pltpu.async_copy(filt.at[idx_v.at[sl]], gft_v.at[sl], gsem))
    for cp in fcps:
        cp.wait()
    pltpu.sync_copy(gft_v, gft_out.at[pl.ds(kw, KW)])


def _make_gather_kernel(qoff):
    def _gather_kernel(xT, ridx, y_out, idx_v, rows0, rows1, gsem, wsem):
        c = lax.axis_index("c")
        s = lax.axis_index("s")
        kw = (s * NUM_CORES + c) * KWQ
        pltpu.sync_copy(ridx.at[pl.ds(qoff + kw, KWQ)], idx_v)

        def idx_sl(n):
            return idx_v.at[pl.ds(n * CH, CH)]

        def y_sl(n):
            return y_out.at[pl.ds(kw + n * CH, CH)]

        pltpu.async_copy(xT.at[idx_sl(0)], rows0, gsem)

        def step(n, buf, other):
            pltpu.make_async_copy(xT.at[idx_sl(n)], buf, gsem).wait()

            @pl.when(n + 1 < NCHQ)
            def _next():
                @pl.when(n >= 1)
                def _drain_prev_write():
                    pltpu.make_async_copy(other, y_sl(n - 1), wsem).wait()
                pltpu.async_copy(xT.at[idx_sl(n + 1)], other, gsem)

            pltpu.async_copy(buf, y_sl(n), wsem)

        def body(m, carry):
            step(m * 2, rows0, rows1)
            step(m * 2 + 1, rows1, rows0)
            return carry
        lax.fori_loop(0, NCHQ // 2, body, 0)

        pltpu.make_async_copy(rows0, y_sl(NCHQ - 2), wsem).wait()
        pltpu.make_async_copy(rows1, y_sl(NCHQ - 1), wsem).wait()
    return _gather_kernel


@jax.jit
def _mapper(x, filt_flat, ridx, fidx):
    mesh = plsc.VectorSubcoreMesh(
        core_axis_name="c", subcore_axis_name="s",
        num_cores=NUM_CORES, num_subcores=NUM_SUBCORES)
    sc_params = pltpu.CompilerParams(use_tc_tiling_on_sc=True)

    gft = functools.partial(
        pl.kernel,
        out_type=jax.ShapeDtypeStruct((K,), jnp.float32),
        mesh=mesh,
        compiler_params=sc_params,
        scratch_types=[
            pltpu.VMEM((KW,), jnp.int32),
            pltpu.VMEM((KW,), jnp.float32),
            pltpu.SemaphoreType.DMA,
        ],
    )(_gft_kernel)(filt_flat, fidx)

    xT = pl.pallas_call(
        _t1_body,
        grid=(D_ALL // CB,),
        in_specs=[pl.BlockSpec((BATCH, CB), lambda i: (0, i))],
        out_specs=pl.BlockSpec((CB, BATCH), lambda i: (i, 0)),
        out_shape=jax.ShapeDtypeStruct((D_ALL, BATCH), jnp.float32),
    )(x)

    ys = []
    for q in range(NQ):
        ys.append(functools.partial(
            pl.kernel,
            out_type=jax.ShapeDtypeStruct((KQ, BATCH), jnp.float32),
            mesh=mesh,
            compiler_params=sc_params,
            scratch_types=[
                pltpu.VMEM((KWQ,), jnp.int32),
                pltpu.VMEM((CH, BATCH), jnp.float32),
                pltpu.VMEM((CH, BATCH), jnp.float32),
                pltpu.SemaphoreType.DMA,
                pltpu.SemaphoreType.DMA,
            ],
        )(_make_gather_kernel(q * KQ))(xT, ridx))

    nqb = KQ // CB
    out = None
    for q in range(NQ):
        if out is None:
            out = pl.pallas_call(
                _t2_body0,
                grid=(nqb,),
                in_specs=[pl.BlockSpec((CB, BATCH), lambda i: (i, 0)),
                          pl.BlockSpec((CB,), lambda i, q=q: (q * nqb + i,))],
                out_specs=pl.BlockSpec((BATCH, CB), lambda i, q=q: (0, q * nqb + i)),
                out_shape=jax.ShapeDtypeStruct((BATCH, K), jnp.float32),
            )(ys[q], gft)
        else:
            out = pl.pallas_call(
                _t2_body1,
                grid=(nqb,),
                in_specs=[pl.BlockSpec((CB, BATCH), lambda i: (i, 0)),
                          pl.BlockSpec((CB,), lambda i, q=q: (q * nqb + i,)),
                          pl.BlockSpec(memory_space=pltpu.MemorySpace.HBM)],
                out_specs=pl.BlockSpec((BATCH, CB), lambda i, q=q: (0, q * nqb + i)),
                out_shape=jax.ShapeDtypeStruct((BATCH, K), jnp.float32),
                input_output_aliases={2: 0},
            )(ys[q], gft, out)
    return out


def kernel(x, filters, ind):
    x = x.reshape(-1, D_ALL)
    ind = ind.astype(jnp.int32)
    ridx = ind.reshape(-1)
    fidx = (ind + jnp.arange(D_COL, dtype=jnp.int32)[:, None] * D_ALL).reshape(-1)
    out2 = _mapper(x, filters.reshape(-1), ridx, fidx)
    return jnp.transpose(out2.reshape(BATCH, D_COL, D_ROW), (0, 2, 1))


# speedup vs baseline: 3.2687x; 1.2205x over previous
import functools

import jax
import jax.numpy as jnp
from jax import lax
from jax.experimental import pallas as pl
from jax.experimental.pallas import tpu as pltpu
from jax.experimental.pallas import tpu_sc as plsc

D_ROW = 4096
D_COL = 64
D_ALL = D_ROW * D_COL
BATCH = 128
K = D_ROW * D_COL
NQ = 4
KQ = K // NQ

NUM_CORES = 2
NUM_SUBCORES = 16
NW = NUM_CORES * NUM_SUBCORES
KW = K // NW
KWQ = KQ // NW
CH = 128
NCHQ = KWQ // CH
FCH = 2048
CB = 2048
IB = 8
JB = 512


def _t1_body(x_ref, o_ref):
    o_ref[...] = x_ref[...].T


def _t2_body0(y_ref, gft_ref, o_ref):
    for u in range(IB):
        o_ref[:, u, :] = y_ref[u].T * gft_ref[u][None, :]


def _t2_body1(y_ref, gft_ref, acc_ref, o_ref):
    del acc_ref
    for u in range(IB):
        o_ref[:, u, :] = y_ref[u].T * gft_ref[u][None, :]


def _gft_kernel(filt, fidx, gft_out, idx_v, gft_v, gsem):
    c = lax.axis_index("c")
    s = lax.axis_index("s")
    kw = (s * NUM_CORES + c) * KW
    pltpu.sync_copy(fidx.at[pl.ds(kw, KW)], idx_v)
    fcps = []
    for q in range(KW // FCH):
        sl = pl.ds(q * FCH, FCH)
        fcps.append(pltpu.async_copy(filt.at[idx_v.at[sl]], gft_v.at[sl], gsem))
    for cp in fcps:
        cp.wait()
    pltpu.sync_copy(gft_v, gft_out.at[pl.ds(kw, KW)])


def _make_gather_kernel(qoff):
    def _gather_kernel(xT, ridx, y_out, idx_v, rows0, rows1, gsem, wsem):
        c = lax.axis_index("c")
        s = lax.axis_index("s")
        kw = (s * NUM_CORES + c) * KWQ
        pltpu.sync_copy(ridx.at[pl.ds(qoff + kw, KWQ)], idx_v)

        def idx_sl(n):
            return idx_v.at[pl.ds(n * CH, CH)]

        def y_sl(n):
            return y_out.at[pl.ds(kw + n * CH, CH)]

        pltpu.async_copy(xT.at[idx_sl(0)], rows0, gsem)

        def step(n, buf, other):
            pltpu.make_async_copy(xT.at[idx_sl(n)], buf, gsem).wait()

            @pl.when(n + 1 < NCHQ)
            def _next():
                @pl.when(n >= 1)
                def _drain_prev_write():
                    pltpu.make_async_copy(other, y_sl(n - 1), wsem).wait()
                pltpu.async_copy(xT.at[idx_sl(n + 1)], other, gsem)

            pltpu.async_copy(buf, y_sl(n), wsem)

        def body(m, carry):
            step(m * 2, rows0, rows1)
            step(m * 2 + 1, rows1, rows0)
            return carry
        lax.fori_loop(0, NCHQ // 2, body, 0)

        pltpu.make_async_copy(rows0, y_sl(NCHQ - 2), wsem).wait()
        pltpu.make_async_copy(rows1, y_sl(NCHQ - 1), wsem).wait()
    return _gather_kernel


@jax.jit
def _mapper(x, filt_flat, ridx, fidx):
    mesh = plsc.VectorSubcoreMesh(
        core_axis_name="c", subcore_axis_name="s",
        num_cores=NUM_CORES, num_subcores=NUM_SUBCORES)
    sc_params = pltpu.CompilerParams(use_tc_tiling_on_sc=True)

    gft = functools.partial(
        pl.kernel,
        out_type=jax.ShapeDtypeStruct((K,), jnp.float32),
        mesh=mesh,
        compiler_params=sc_params,
        scratch_types=[
            pltpu.VMEM((KW,), jnp.int32),
            pltpu.VMEM((KW,), jnp.float32),
            pltpu.SemaphoreType.DMA,
        ],
    )(_gft_kernel)(filt_flat, fidx)

    xT = pl.pallas_call(
        _t1_body,
        grid=(D_ALL // CB,),
        in_specs=[pl.BlockSpec((BATCH, CB), lambda i: (0, i))],
        out_specs=pl.BlockSpec((CB, BATCH), lambda i: (i, 0)),
        out_shape=jax.ShapeDtypeStruct((D_ALL, BATCH), jnp.float32),
    )(x)

    ys = []
    for q in range(NQ):
        ys.append(functools.partial(
            pl.kernel,
            out_type=jax.ShapeDtypeStruct((KQ, BATCH), jnp.float32),
            mesh=mesh,
            compiler_params=sc_params,
            scratch_types=[
                pltpu.VMEM((KWQ,), jnp.int32),
                pltpu.VMEM((CH, BATCH), jnp.float32),
                pltpu.VMEM((CH, BATCH), jnp.float32),
                pltpu.SemaphoreType.DMA,
                pltpu.SemaphoreType.DMA,
            ],
        )(_make_gather_kernel(q * KQ))(xT, ridx))

    nqi = KQ // D_ROW
    gft2 = gft.reshape(D_COL, D_ROW)
    out = None
    for q in range(NQ):
        y3 = ys[q].reshape(nqi, D_ROW, BATCH)
        grid = (nqi // IB, D_ROW // JB)
        in_specs = [pl.BlockSpec((IB, JB, BATCH), lambda ib, jb: (ib, jb, 0)),
                    pl.BlockSpec((IB, JB), lambda ib, jb, q=q: (q * (nqi // IB) + ib, jb))]
        out_spec = pl.BlockSpec((BATCH, IB, JB),
                                lambda ib, jb, q=q: (0, q * (nqi // IB) + ib, jb))
        oshape = jax.ShapeDtypeStruct((BATCH, D_COL, D_ROW), jnp.float32)
        if out is None:
            out = pl.pallas_call(
                _t2_body0, grid=grid, in_specs=in_specs,
                out_specs=out_spec, out_shape=oshape,
            )(y3, gft2)
        else:
            out = pl.pallas_call(
                _t2_body1, grid=grid,
                in_specs=in_specs + [pl.BlockSpec(memory_space=pltpu.MemorySpace.HBM)],
                out_specs=out_spec, out_shape=oshape,
                input_output_aliases={2: 0},
            )(y3, gft2, out)
    return out


def kernel(x, filters, ind):
    x = x.reshape(-1, D_ALL)
    ind = ind.astype(jnp.int32)
    ridx = ind.reshape(-1)
    fidx = (ind + jnp.arange(D_COL, dtype=jnp.int32)[:, None] * D_ALL).reshape(-1)
    out3 = _mapper(x, filters.reshape(-1), ridx, fidx)
    return jnp.transpose(out3, (0, 2, 1))
